# baseline (speedup 1.0000x reference)
import jax
import jax.numpy as jnp
from jax.experimental import pallas as pl
from jax.experimental.pallas import tpu as pltpu

_N = 1024
_P1 = 160
_P2 = 80
_BG_AREA = 321 ** 2 - 161 ** 2
_FRONT_DIV = (161 ** 2) * 1.8
_SCALE = float(_BG_AREA / _FRONT_DIV)


def _cfar_kernel(x_ref, o_ref):
    x = x_ref[0, 0].astype(jnp.bfloat16)
    i = jax.lax.broadcasted_iota(jnp.int32, (_N, _N), 0)
    j = jax.lax.broadcasted_iota(jnp.int32, (_N, _N), 1)
    d = jnp.abs(i - j)
    b1 = jnp.where(d <= _P1, 1.0, 0.0).astype(jnp.bfloat16)
    b2 = jnp.where(d <= _P2, 1.0, 0.0).astype(jnp.bfloat16)
    y1 = jnp.dot(b1, x, preferred_element_type=jnp.float32).astype(jnp.bfloat16)
    y2 = jnp.dot(b2, x, preferred_element_type=jnp.float32).astype(jnp.bfloat16)
    allsum = jnp.dot(y1, b1, preferred_element_type=jnp.float32)
    front = jnp.dot(y2, b2, preferred_element_type=jnp.float32)
    o_ref[0, 0] = front * (_SCALE / (allsum - front))


def kernel(x):
    return pl.pallas_call(
        _cfar_kernel,
        out_shape=jax.ShapeDtypeStruct((4, 1, _N, _N), jnp.float32),
        grid=(4,),
        in_specs=[pl.BlockSpec((1, 1, _N, _N), lambda b: (b, 0, 0, 0))],
        out_specs=pl.BlockSpec((1, 1, _N, _N), lambda b: (b, 0, 0, 0)),
        compiler_params=pltpu.CompilerParams(
            dimension_semantics=("parallel",),
            vmem_limit_bytes=100 * 1024 * 1024,
        ),
        name="cfar_banded_matmul",
    )(x)


# speedup vs baseline: 1.2507x; 1.2507x over previous
import jax
import jax.numpy as jnp
from jax.experimental import pallas as pl
from jax.experimental.pallas import tpu as pltpu

_N = 1024
_G = 2
_P1 = 160
_P2 = 80
_BG_AREA = 321 ** 2 - 161 ** 2
_FRONT_DIV = (161 ** 2) * 1.8
_SCALE = float(_BG_AREA / _FRONT_DIV)


def _row_boxdiff(s, s_down1, p):
    hi = jnp.concatenate(
        [s[p:], jnp.broadcast_to(s[_N - 1:_N], (p, _N))], axis=0)
    lo = jnp.concatenate(
        [jnp.zeros((p, _N), jnp.float32), s_down1[:_N - p]], axis=0)
    return hi - lo


def _col_boxdiff(s, p):
    hi = jnp.concatenate(
        [s[:, p:], jnp.broadcast_to(s[:, _N - 1:_N], (_N, p))], axis=1)
    lo = jnp.concatenate(
        [jnp.zeros((_N, p + 1), jnp.float32), s[:, :_N - p - 1]], axis=1)
    return hi - lo


def _cfar_kernel(x_ref, o_ref):
    i = jax.lax.broadcasted_iota(jnp.int32, (_N, _N), 0)
    j = jax.lax.broadcasted_iota(jnp.int32, (_N, _N), 1)
    ltri = jnp.where(i >= j, 1.0, 0.0)
    utri = jnp.where(i <= j, 1.0, 0.0)
    for g in range(_G):
        x = x_ref[g, 0]
        c = jnp.dot(ltri, x, preferred_element_type=jnp.float32)
        s = jnp.dot(c, utri, preferred_element_type=jnp.float32)
        s_down1 = jnp.concatenate(
            [jnp.zeros((1, _N), jnp.float32), s[:_N - 1]], axis=0)
        allsum = _col_boxdiff(_row_boxdiff(s, s_down1, _P1), _P1)
        front = _col_boxdiff(_row_boxdiff(s, s_down1, _P2), _P2)
        o_ref[g, 0] = front * (_SCALE / (allsum - front))


def kernel(x):
    return pl.pallas_call(
        _cfar_kernel,
        out_shape=jax.ShapeDtypeStruct((4, 1, _N, _N), jnp.float32),
        grid=(4 // _G,),
        in_specs=[pl.BlockSpec((_G, 1, _N, _N), lambda b: (b, 0, 0, 0))],
        out_specs=pl.BlockSpec((_G, 1, _N, _N), lambda b: (b, 0, 0, 0)),
        compiler_params=pltpu.CompilerParams(
            dimension_semantics=("parallel",),
            vmem_limit_bytes=100 * 1024 * 1024,
        ),
        name="cfar_integral_image",
    )(x)


# speedup vs baseline: 1.2756x; 1.0199x over previous
import jax
import jax.numpy as jnp
from jax.experimental import pallas as pl
from jax.experimental.pallas import tpu as pltpu

_N = 1024
_G = 2
_B = 256
_NB = _N // _B
_P1 = 160
_P2 = 80
_BG_AREA = 321 ** 2 - 161 ** 2
_FRONT_DIV = (161 ** 2) * 1.8
_SCALE = float(_BG_AREA / _FRONT_DIV)


def _row_boxdiff(s, s_down1, p):
    hi = jnp.concatenate(
        [s[p:], jnp.broadcast_to(s[_N - 1:_N], (p, _N))], axis=0)
    lo = jnp.concatenate(
        [jnp.zeros((p, _N), jnp.float32), s_down1[:_N - p]], axis=0)
    return hi - lo


def _col_boxdiff(s, p):
    hi = jnp.concatenate(
        [s[:, p:], jnp.broadcast_to(s[:, _N - 1:_N], (_N, p))], axis=1)
    lo = jnp.concatenate(
        [jnp.zeros((_N, p + 1), jnp.float32), s[:, :_N - p - 1]], axis=1)
    return hi - lo


def _blocked_col_cumsum(x, ltri):
    locs = [
        jnp.dot(ltri, x[t * _B:(t + 1) * _B],
                preferred_element_type=jnp.float32)
        for t in range(_NB)
    ]
    parts = [locs[0]]
    for t in range(1, _NB):
        parts.append(locs[t] + parts[t - 1][_B - 1:_B])
    return jnp.concatenate(parts, axis=0)


def _blocked_row_cumsum(c, utri):
    locs = [
        jnp.dot(c[:, t * _B:(t + 1) * _B], utri,
                preferred_element_type=jnp.float32)
        for t in range(_NB)
    ]
    parts = [locs[0]]
    for t in range(1, _NB):
        parts.append(locs[t] + parts[t - 1][:, _B - 1:_B])
    return jnp.concatenate(parts, axis=1)


def _cfar_kernel(x_ref, o_ref):
    i = jax.lax.broadcasted_iota(jnp.int32, (_B, _B), 0)
    j = jax.lax.broadcasted_iota(jnp.int32, (_B, _B), 1)
    ltri = jnp.where(i >= j, 1.0, 0.0)
    utri = jnp.where(i <= j, 1.0, 0.0)
    for g in range(_G):
        x = x_ref[g, 0]
        c = _blocked_col_cumsum(x, ltri)
        s = _blocked_row_cumsum(c, utri)
        s_down1 = jnp.concatenate(
            [jnp.zeros((1, _N), jnp.float32), s[:_N - 1]], axis=0)
        allsum = _col_boxdiff(_row_boxdiff(s, s_down1, _P1), _P1)
        front = _col_boxdiff(_row_boxdiff(s, s_down1, _P2), _P2)
        o_ref[g, 0] = front * (_SCALE / (allsum - front))


def kernel(x):
    return pl.pallas_call(
        _cfar_kernel,
        out_shape=jax.ShapeDtypeStruct((4, 1, _N, _N), jnp.float32),
        grid=(4 // _G,),
        in_specs=[pl.BlockSpec((_G, 1, _N, _N), lambda b: (b, 0, 0, 0))],
        out_specs=pl.BlockSpec((_G, 1, _N, _N), lambda b: (b, 0, 0, 0)),
        compiler_params=pltpu.CompilerParams(
            dimension_semantics=("parallel",),
            vmem_limit_bytes=100 * 1024 * 1024,
        ),
        name="cfar_integral_image",
    )(x)


# speedup vs baseline: 1.4074x; 1.1033x over previous
import jax
import jax.numpy as jnp
from jax.experimental import pallas as pl
from jax.experimental.pallas import tpu as pltpu

_N = 1024
_G = 1
_B = 256
_NB = _N // _B
_P1 = 160
_P2 = 80
_BG_AREA = 321 ** 2 - 161 ** 2
_FRONT_DIV = (161 ** 2) * 1.8
_SCALE = float(_BG_AREA / _FRONT_DIV)


def _row_boxdiff(s, s_down1, p):
    hi = jnp.concatenate(
        [s[p:], jnp.broadcast_to(s[_N - 1:_N], (p, _N))], axis=0)
    lo = jnp.concatenate(
        [jnp.zeros((p, _N), jnp.float32), s_down1[:_N - p]], axis=0)
    return hi - lo


def _col_boxdiff(s, p):
    hi = jnp.concatenate(
        [s[:, p:], jnp.broadcast_to(s[:, _N - 1:_N], (_N, p))], axis=1)
    lo = jnp.concatenate(
        [jnp.zeros((_N, p + 1), jnp.float32), s[:, :_N - p - 1]], axis=1)
    return hi - lo


def _blocked_col_cumsum(x, ltri):
    locs = [
        jnp.dot(ltri, x[t * _B:(t + 1) * _B],
                preferred_element_type=jnp.float32)
        for t in range(_NB)
    ]
    parts = [locs[0]]
    for t in range(1, _NB):
        parts.append(locs[t] + parts[t - 1][_B - 1:_B])
    return jnp.concatenate(parts, axis=0)


def _blocked_row_cumsum(c, utri):
    locs = [
        jnp.dot(c[:, t * _B:(t + 1) * _B], utri,
                preferred_element_type=jnp.float32)
        for t in range(_NB)
    ]
    parts = [locs[0]]
    for t in range(1, _NB):
        parts.append(locs[t] + parts[t - 1][:, _B - 1:_B])
    return jnp.concatenate(parts, axis=1)


def _cfar_kernel(x_ref, o_ref):
    i = jax.lax.broadcasted_iota(jnp.int32, (_B, _B), 0)
    j = jax.lax.broadcasted_iota(jnp.int32, (_B, _B), 1)
    ltri = jnp.where(i >= j, 1.0, 0.0)
    utri = jnp.where(i <= j, 1.0, 0.0)
    for g in range(_G):
        x = x_ref[g, 0]
        c = _blocked_col_cumsum(x, ltri)
        s = _blocked_row_cumsum(c, utri)
        s_down1 = jnp.concatenate(
            [jnp.zeros((1, _N), jnp.float32), s[:_N - 1]], axis=0)
        allsum = _col_boxdiff(_row_boxdiff(s, s_down1, _P1), _P1)
        front = _col_boxdiff(_row_boxdiff(s, s_down1, _P2), _P2)
        o_ref[g, 0] = front * (_SCALE / (allsum - front))


def kernel(x):
    return pl.pallas_call(
        _cfar_kernel,
        out_shape=jax.ShapeDtypeStruct((4, 1, _N, _N), jnp.float32),
        grid=(4 // _G,),
        in_specs=[pl.BlockSpec((_G, 1, _N, _N), lambda b: (b, 0, 0, 0))],
        out_specs=pl.BlockSpec((_G, 1, _N, _N), lambda b: (b, 0, 0, 0)),
        compiler_params=pltpu.CompilerParams(
            dimension_semantics=("parallel",),
            vmem_limit_bytes=100 * 1024 * 1024,
        ),
        name="cfar_integral_image",
    )(x)


# speedup vs baseline: 1.7983x; 1.2778x over previous
import jax
import jax.numpy as jnp
from jax.experimental import pallas as pl
from jax.experimental.pallas import tpu as pltpu

_N = 1024
_G = 1
_B = 256
_NB = _N // _B
_P1 = 160
_P2 = 80
_BG_AREA = 321 ** 2 - 161 ** 2
_FRONT_DIV = (161 ** 2) * 1.8
_SCALE = float(_BG_AREA / _FRONT_DIV)


def _row_boxdiff(s, s_down1, p):
    hi = jnp.concatenate(
        [s[p:], jnp.broadcast_to(s[_N - 1:_N], (p, _N))], axis=0)
    lo = jnp.concatenate(
        [jnp.zeros((p, _N), jnp.float32), s_down1[:_N - p]], axis=0)
    return hi - lo


def _col_boxdiff(s, p):
    hi = jnp.concatenate(
        [s[:, p:], jnp.broadcast_to(s[:, _N - 1:_N], (_N, p))], axis=1)
    lo = jnp.concatenate(
        [jnp.zeros((_N, p + 1), jnp.float32), s[:, :_N - p - 1]], axis=1)
    return hi - lo


def _blocked_col_cumsum(x, ltri):
    locs = [
        jnp.dot(ltri, x[t * _B:(t + 1) * _B],
                preferred_element_type=jnp.float32)
        for t in range(_NB)
    ]
    parts = [locs[0]]
    for t in range(1, _NB):
        parts.append(locs[t] + parts[t - 1][_B - 1:_B])
    return jnp.concatenate(parts, axis=0)


def _blocked_row_cumsum(c, utri):
    locs = [
        jnp.dot(c[:, t * _B:(t + 1) * _B], utri,
                preferred_element_type=jnp.float32)
        for t in range(_NB)
    ]
    parts = [locs[0]]
    for t in range(1, _NB):
        parts.append(locs[t] + parts[t - 1][:, _B - 1:_B])
    return jnp.concatenate(parts, axis=1)


def _cfar_kernel(x_ref, o_ref):
    i = jax.lax.broadcasted_iota(jnp.int32, (_B, _B), 0)
    j = jax.lax.broadcasted_iota(jnp.int32, (_B, _B), 1)
    ltri = jnp.where(i >= j, 1.0, 0.0)
    utri = jnp.where(i <= j, 1.0, 0.0)
    ib = jax.lax.broadcasted_iota(jnp.int32, (_N, _N), 0)
    jb = jax.lax.broadcasted_iota(jnp.int32, (_N, _N), 1)
    band1 = jnp.where(jnp.abs(ib - jb) <= _P1, 1.0, 0.0)
    for g in range(_G):
        x = x_ref[g, 0]
        c = _blocked_col_cumsum(x, ltri)
        s = _blocked_row_cumsum(c, utri)
        s_down1 = jnp.concatenate(
            [jnp.zeros((1, _N), jnp.float32), s[:_N - 1]], axis=0)
        a_cols = jnp.dot(c, band1, preferred_element_type=jnp.float32)
        allsum = _row_boxdiff(a_cols, jnp.concatenate(
            [jnp.zeros((1, _N), jnp.float32), a_cols[:_N - 1]], axis=0), _P1)
        front = _col_boxdiff(_row_boxdiff(s, s_down1, _P2), _P2)
        o_ref[g, 0] = front * (_SCALE / (allsum - front))


def kernel(x):
    return pl.pallas_call(
        _cfar_kernel,
        out_shape=jax.ShapeDtypeStruct((4, 1, _N, _N), jnp.float32),
        grid=(4 // _G,),
        in_specs=[pl.BlockSpec((_G, 1, _N, _N), lambda b: (b, 0, 0, 0))],
        out_specs=pl.BlockSpec((_G, 1, _N, _N), lambda b: (b, 0, 0, 0)),
        compiler_params=pltpu.CompilerParams(
            dimension_semantics=("parallel",),
            vmem_limit_bytes=100 * 1024 * 1024,
        ),
        name="cfar_integral_image",
    )(x)


# speedup vs baseline: 1.9097x; 1.0619x over previous
import jax
import jax.numpy as jnp
from jax.experimental import pallas as pl
from jax.experimental.pallas import tpu as pltpu

_N = 1024
_G = 1
_B = 256
_NB = _N // _B
_P1 = 160
_P2 = 80
_BG_AREA = 321 ** 2 - 161 ** 2
_FRONT_DIV = (161 ** 2) * 1.8
_SCALE = float(_BG_AREA / _FRONT_DIV)


def _row_boxdiff(s, s_down1, p):
    hi = jnp.concatenate(
        [s[p:], jnp.broadcast_to(s[_N - 1:_N], (p, _N))], axis=0)
    lo = jnp.concatenate(
        [jnp.zeros((p, _N), jnp.float32), s_down1[:_N - p]], axis=0)
    return hi - lo


def _col_boxdiff(s, p):
    hi = jnp.concatenate(
        [s[:, p:], jnp.broadcast_to(s[:, _N - 1:_N], (_N, p))], axis=1)
    lo = jnp.concatenate(
        [jnp.zeros((_N, p + 1), jnp.float32), s[:, :_N - p - 1]], axis=1)
    return hi - lo


def _blocked_col_cumsum(x, ltri):
    locs = [
        jnp.dot(ltri, x[t * _B:(t + 1) * _B],
                preferred_element_type=jnp.float32)
        for t in range(_NB)
    ]
    parts = [locs[0]]
    for t in range(1, _NB):
        parts.append(locs[t] + parts[t - 1][_B - 1:_B])
    return jnp.concatenate(parts, axis=0)


def _blocked_row_cumsum(c, utri):
    locs = [
        jnp.dot(c[:, t * _B:(t + 1) * _B], utri,
                preferred_element_type=jnp.float32)
        for t in range(_NB)
    ]
    parts = [locs[0]]
    for t in range(1, _NB):
        parts.append(locs[t] + parts[t - 1][:, _B - 1:_B])
    return jnp.concatenate(parts, axis=1)


def _banded_colsum(c):
    outs = []
    for t in range(_N // 256):
        lo = max(0, ((256 * t - _P1) // 128) * 128)
        hi = min(_N, ((256 * t + 255 + _P1) // 128 + 1) * 128)
        ib = jax.lax.broadcasted_iota(jnp.int32, (hi - lo, 256), 0) + lo
        jb = jax.lax.broadcasted_iota(jnp.int32, (hi - lo, 256), 1) + 256 * t
        bslab = jnp.where(jnp.abs(ib - jb) <= _P1, 1.0, 0.0)
        outs.append(jnp.dot(c[:, lo:hi], bslab,
                            preferred_element_type=jnp.float32))
    return jnp.concatenate(outs, axis=1)


def _cfar_kernel(x_ref, o_ref):
    i = jax.lax.broadcasted_iota(jnp.int32, (_B, _B), 0)
    j = jax.lax.broadcasted_iota(jnp.int32, (_B, _B), 1)
    ltri = jnp.where(i >= j, 1.0, 0.0)
    utri = jnp.where(i <= j, 1.0, 0.0)

    for g in range(_G):
        x = x_ref[g, 0]
        c = _blocked_col_cumsum(x, ltri)
        s = _blocked_row_cumsum(c, utri)
        s_down1 = jnp.concatenate(
            [jnp.zeros((1, _N), jnp.float32), s[:_N - 1]], axis=0)
        a_cols = _banded_colsum(c)
        allsum = _row_boxdiff(a_cols, jnp.concatenate(
            [jnp.zeros((1, _N), jnp.float32), a_cols[:_N - 1]], axis=0), _P1)
        front = _col_boxdiff(_row_boxdiff(s, s_down1, _P2), _P2)
        o_ref[g, 0] = front * (_SCALE / (allsum - front))


def kernel(x):
    return pl.pallas_call(
        _cfar_kernel,
        out_shape=jax.ShapeDtypeStruct((4, 1, _N, _N), jnp.float32),
        grid=(4 // _G,),
        in_specs=[pl.BlockSpec((_G, 1, _N, _N), lambda b: (b, 0, 0, 0))],
        out_specs=pl.BlockSpec((_G, 1, _N, _N), lambda b: (b, 0, 0, 0)),
        compiler_params=pltpu.CompilerParams(
            dimension_semantics=("parallel",),
            vmem_limit_bytes=100 * 1024 * 1024,
        ),
        name="cfar_integral_image",
    )(x)
